# baseline (speedup 1.0000x reference)
import functools

import jax
import jax.numpy as jnp
from jax import lax
from jax.experimental import pallas as pl
from jax.experimental.pallas import tpu as pltpu
from jax.experimental.pallas import tpu_sc as plsc

N_NODES = 10000
N_EDGES = 320000
F = 128
B = 16
BA = 24
NC = 2
NS = 16
EPT = N_EDGES // (NC * NS)
CHUNK = 80
NCHUNK = EPT // CHUNK
BLK = 8
BLKE = BLK * CHUNK
NBLK_FULL = NCHUNK // BLK
BLK_REM = NCHUNK - NBLK_FULL * BLK
WRPT = 624


def _sc_nbr_sum(x, edge_index):
    mesh = plsc.VectorSubcoreMesh(core_axis_name="c", subcore_axis_name="s")

    @functools.partial(
        pl.kernel,
        out_type=jax.ShapeDtypeStruct((NC, N_NODES, F), jnp.float32),
        mesh=mesh,
        compiler_params=pltpu.CompilerParams(use_tc_tiling_on_sc=False),
        scratch_types=(
            pltpu.VMEM_SHARED((N_NODES, F), jnp.float32),
            pltpu.VMEM((2, BLKE), jnp.int32),
            pltpu.VMEM((2, BLKE), jnp.int32),
            pltpu.VMEM((2, CHUNK, F), jnp.float32),
            pltpu.SemaphoreType.DMA,
            pltpu.SemaphoreType.DMA,
        ),
    )
    def k(x_hbm, ei_hbm, out_hbm, acc, sblk, nblk, rows, gsem0, gsem1):
        gsems = (gsem0, gsem1)
        c = lax.axis_index("c")
        s = lax.axis_index("s")
        ebase = (c * NS + s) * EPT

        zv = jnp.zeros((16,), jnp.float32)

        @pl.loop(0, CHUNK)
        def _(r):
            for j in range(F // 16):
                rows[0, r, pl.ds(j * 16, 16)] = zv

        done = 0
        while done < WRPT:
            n = min(CHUNK, WRPT - done)
            pltpu.sync_copy(rows.at[0, pl.ds(0, n)],
                            acc.at[pl.ds(s * WRPT + done, n)])
            done += n

        @pl.when(s == 0)
        def _():
            rem = N_NODES - NS * WRPT
            pltpu.sync_copy(rows.at[0, pl.ds(0, rem)],
                            acc.at[pl.ds(NS * WRPT, rem)])
        plsc.subcore_barrier()

        def load_blk(n, nedges, buf):
            off = ebase + n * BLKE
            pltpu.sync_copy(ei_hbm.at[0, pl.ds(off, nedges)],
                            sblk.at[buf, pl.ds(0, nedges)])
            pltpu.sync_copy(ei_hbm.at[1, pl.ds(off, nedges)],
                            nblk.at[buf, pl.ds(0, nedges)])

        def sidx(j):
            return sblk.at[lax.rem(j // BLK, 2),
                           pl.ds(lax.rem(j, BLK) * CHUNK, CHUNK)]

        def nidx(j):
            return nblk.at[lax.rem(j // BLK, 2),
                           pl.ds(lax.rem(j, BLK) * CHUNK, CHUNK)]

        def start_fetch(j, b):
            pltpu.async_copy(x_hbm.at[nidx(j)], rows.at[b], gsems[b])

        def process(j, b):
            pltpu.make_async_copy(x_hbm.at[nidx(j)], rows.at[b],
                                  gsems[b]).wait()
            pltpu.sync_copy(rows.at[b], acc.at[sidx(j)], add=True)

            @pl.when(lax.rem(j, BLK) == 0)
            def _():
                nxt = j // BLK + 1

                @pl.when(nxt < NBLK_FULL)
                def _():
                    load_blk(nxt, BLKE, lax.rem(nxt, 2))

                @pl.when(nxt == NBLK_FULL)
                def _():
                    load_blk(nxt, BLK_REM * CHUNK, lax.rem(nxt, 2))

            @pl.when(j + 2 < NCHUNK)
            def _():
                start_fetch(j + 2, b)

        load_blk(0, BLKE, 0)
        start_fetch(0, 0)
        start_fetch(1, 1)

        @pl.loop(0, NCHUNK - 1, step=2)
        def _(i):
            for b in range(2):
                process(i + b, b)

        process(NCHUNK - 1, 0)
        plsc.subcore_barrier()

        wsub = pl.ds(s * WRPT, WRPT)
        pltpu.sync_copy(acc.at[wsub], out_hbm.at[c, wsub])

        @pl.when(s == 0)
        def _():
            rsub = pl.ds(NS * WRPT, N_NODES - NS * WRPT)
            pltpu.sync_copy(acc.at[rsub], out_hbm.at[c, rsub])

    return k(x, edge_index)


def _sc_ed_sum(edge_index, edge_attr, ones_pad, zeros_ba):
    mesh = plsc.VectorSubcoreMesh(core_axis_name="c", subcore_axis_name="s")

    @functools.partial(
        pl.kernel,
        out_type=jax.ShapeDtypeStruct((NC, N_NODES, BA), jnp.float32),
        mesh=mesh,
        compiler_params=pltpu.CompilerParams(use_tc_tiling_on_sc=False),
        scratch_types=(
            pltpu.VMEM_SHARED((N_NODES, BA), jnp.float32),
            pltpu.VMEM((2, BLKE), jnp.int32),
            pltpu.VMEM((2, CHUNK, BA), jnp.float32),
            pltpu.SemaphoreType.DMA,
            pltpu.SemaphoreType.DMA,
        ),
    )
    def k(ei_hbm, attr_hbm, op_hbm, z_hbm, out_hbm, acc, sblk, attr,
          asem0, asem1):
        asems = (asem0, asem1)
        c = lax.axis_index("c")
        s = lax.axis_index("s")
        ebase = (c * NS + s) * EPT

        pltpu.sync_copy(z_hbm.at[pl.ds(s * WRPT, WRPT)],
                        acc.at[pl.ds(s * WRPT, WRPT)])

        @pl.when(s == 0)
        def _():
            rem = N_NODES - NS * WRPT
            pltpu.sync_copy(z_hbm.at[pl.ds(NS * WRPT, rem)],
                            acc.at[pl.ds(NS * WRPT, rem)])

        for b in range(2):
            pltpu.sync_copy(op_hbm, attr.at[b, :, pl.ds(B, BA - B)])
        plsc.subcore_barrier()

        def load_blk(n, nedges, buf):
            off = ebase + n * BLKE
            pltpu.sync_copy(ei_hbm.at[0, pl.ds(off, nedges)],
                            sblk.at[buf, pl.ds(0, nedges)])

        def sidx(j):
            return sblk.at[lax.rem(j // BLK, 2),
                           pl.ds(lax.rem(j, BLK) * CHUNK, CHUNK)]

        def start_fetch(j, b):
            off = ebase + j * CHUNK
            pltpu.async_copy(attr_hbm.at[pl.ds(off, CHUNK)],
                             attr.at[b, :, pl.ds(0, B)], asems[b])

        def process(j, b):
            off = ebase + j * CHUNK
            pltpu.make_async_copy(attr_hbm.at[pl.ds(off, CHUNK)],
                                  attr.at[b, :, pl.ds(0, B)],
                                  asems[b]).wait()
            pltpu.sync_copy(attr.at[b], acc.at[sidx(j)], add=True)

            @pl.when(lax.rem(j, BLK) == 0)
            def _():
                nxt = j // BLK + 1

                @pl.when(nxt < NBLK_FULL)
                def _():
                    load_blk(nxt, BLKE, lax.rem(nxt, 2))

                @pl.when(nxt == NBLK_FULL)
                def _():
                    load_blk(nxt, BLK_REM * CHUNK, lax.rem(nxt, 2))

            @pl.when(j + 2 < NCHUNK)
            def _():
                start_fetch(j + 2, b)

        load_blk(0, BLKE, 0)
        start_fetch(0, 0)
        start_fetch(1, 1)

        @pl.loop(0, NCHUNK - 1, step=2)
        def _(i):
            for b in range(2):
                process(i + b, b)

        process(NCHUNK - 1, 0)
        plsc.subcore_barrier()

        wsub = pl.ds(s * WRPT, WRPT)
        pltpu.sync_copy(acc.at[wsub], out_hbm.at[c, wsub])

        @pl.when(s == 0)
        def _():
            rsub = pl.ds(NS * WRPT, N_NODES - NS * WRPT)
            pltpu.sync_copy(acc.at[rsub], out_hbm.at[c, rsub])

    return k(edge_index, edge_attr, ones_pad, zeros_ba)


def _tc_combine(x, nbr_part, ed_part, w_s, w_nx, w_ne):
    R = 1000

    def body(x_ref, nbr_ref, ed_ref, ws_ref, wnx_ref, wne_ref, out_ref):
        xv = x_ref[...]
        deg = ed_ref[0, :, B:B + 1] + ed_ref[1, :, B:B + 1]
        y = deg * xv + nbr_ref[0] + nbr_ref[1]
        e = ed_ref[0, :, :B] + ed_ref[1, :, :B]
        acc = jnp.dot(xv, ws_ref[...], preferred_element_type=jnp.float32)
        acc = acc + jnp.dot(y, wnx_ref[...],
                            preferred_element_type=jnp.float32)
        acc = acc + jnp.dot(e, wne_ref[...],
                            preferred_element_type=jnp.float32)
        out_ref[...] = acc

    return pl.pallas_call(
        body,
        grid=(N_NODES // R,),
        in_specs=[
            pl.BlockSpec((R, F), lambda i: (i, 0)),
            pl.BlockSpec((NC, R, F), lambda i: (0, i, 0)),
            pl.BlockSpec((NC, R, BA), lambda i: (0, i, 0)),
            pl.BlockSpec((F, F), lambda i: (0, 0)),
            pl.BlockSpec((F, F), lambda i: (0, 0)),
            pl.BlockSpec((B, F), lambda i: (0, 0)),
        ],
        out_specs=pl.BlockSpec((R, F), lambda i: (i, 0)),
        out_shape=jax.ShapeDtypeStruct((N_NODES, F), jnp.float32),
    )(x, nbr_part, ed_part, w_s, w_nx, w_ne)


def kernel(x, edge_index, edge_attr, w_s, w_n):
    ones_pad = jnp.ones((CHUNK, BA - B), jnp.float32)
    zeros_ba = jnp.zeros((N_NODES, BA), jnp.float32)
    w_nx = w_n[:F]
    w_ne = w_n[F:]
    nbr_part = _sc_nbr_sum(x, edge_index)
    ed_part = _sc_ed_sum(edge_index, edge_attr, ones_pad, zeros_ba)
    return _tc_combine(x, nbr_part, ed_part, w_s, w_nx, w_ne)
